# baseline (speedup 1.0000x reference)
import jax
import jax.numpy as jnp
from jax.experimental import pallas as pl

_D = 128
_BLOCK = 10000


def _gclstm_body(x_ref, w_ref, b_ref, wco_ref, fct_ref, fcb_ref, o_ref):
    xw = jnp.dot(x_ref[...], w_ref[...], preferred_element_type=jnp.float32)
    xw = xw + b_ref[...]
    gi = jax.nn.sigmoid(xw[:, :_D])
    gt = jnp.tanh(xw[:, _D:2 * _D])
    cn = gi * gt
    go = jax.nn.sigmoid(xw[:, 2 * _D:] + wco_ref[...] * cn)
    hn = go * jnp.tanh(cn)
    o_ref[...] = jnp.dot(hn, fct_ref[...],
                         preferred_element_type=jnp.float32) + fcb_ref[...]


def kernel(x, edge_index, batch, W_i, W_f, W_c, W_o, Th_i, Th_f, Th_c, Th_o,
           bch_i, bch_f, bch_c, bch_o, w_ci, w_cf, w_co, b_i, b_f, b_c, b_o,
           fc_w, fc_b):
    n = x.shape[0]
    w_cat = jnp.concatenate([W_i, W_c, W_o], axis=1)
    b_cat = jnp.concatenate([(bch_i + b_i[0])[None, :],
                             (bch_c + b_c[0])[None, :],
                             (bch_o + b_o[0])[None, :]], axis=1)
    fc_wt = fc_w.T
    fc_b2 = fc_b[None, :]

    grid = (n // _BLOCK,)
    return pl.pallas_call(
        _gclstm_body,
        grid=grid,
        in_specs=[
            pl.BlockSpec((_BLOCK, _D), lambda i: (i, 0)),
            pl.BlockSpec((_D, 3 * _D), lambda i: (0, 0)),
            pl.BlockSpec((1, 3 * _D), lambda i: (0, 0)),
            pl.BlockSpec((1, _D), lambda i: (0, 0)),
            pl.BlockSpec((_D, _D), lambda i: (0, 0)),
            pl.BlockSpec((1, _D), lambda i: (0, 0)),
        ],
        out_specs=pl.BlockSpec((_BLOCK, _D), lambda i: (i, 0)),
        out_shape=jax.ShapeDtypeStruct((n, _D), jnp.float32),
    )(x, w_cat, b_cat, w_co, fc_wt, fc_b2)


# speedup vs baseline: 1.2022x; 1.2022x over previous
import jax
import jax.numpy as jnp
from jax.experimental import pallas as pl

_D = 128
_BLOCK = 5000


def _gclstm_body(x_ref, wi_ref, wc_ref, wo_ref, bi_ref, bc_ref, bo_ref,
                 wco_ref, fcw_ref, fcb_ref, o_ref):
    x = x_ref[...]
    xi = jnp.dot(x, wi_ref[...], preferred_element_type=jnp.float32)
    xc = jnp.dot(x, wc_ref[...], preferred_element_type=jnp.float32)
    xo = jnp.dot(x, wo_ref[...], preferred_element_type=jnp.float32)
    gi = jax.nn.sigmoid(xi + bi_ref[...])
    gt = jnp.tanh(xc + bc_ref[...])
    cn = gi * gt
    go = jax.nn.sigmoid(xo + bo_ref[...] + wco_ref[...] * cn)
    hn = go * jnp.tanh(cn)
    out = jax.lax.dot_general(hn, fcw_ref[...],
                              dimension_numbers=(((1,), (1,)), ((), ())),
                              preferred_element_type=jnp.float32)
    o_ref[...] = out + fcb_ref[...]


def kernel(x, edge_index, batch, W_i, W_f, W_c, W_o, Th_i, Th_f, Th_c, Th_o,
           bch_i, bch_f, bch_c, bch_o, w_ci, w_cf, w_co, b_i, b_f, b_c, b_o,
           fc_w, fc_b):
    n = x.shape[0]
    bi = (bch_i + b_i[0])[None, :]
    bc = (bch_c + b_c[0])[None, :]
    bo = (bch_o + b_o[0])[None, :]
    fcb = fc_b[None, :]

    full = lambda shape: pl.BlockSpec(shape, lambda i: (0,) * len(shape))
    return pl.pallas_call(
        _gclstm_body,
        grid=(n // _BLOCK,),
        in_specs=[
            pl.BlockSpec((_BLOCK, _D), lambda i: (i, 0)),
            full((_D, _D)), full((_D, _D)), full((_D, _D)),
            full((1, _D)), full((1, _D)), full((1, _D)),
            full((1, _D)), full((_D, _D)), full((1, _D)),
        ],
        out_specs=pl.BlockSpec((_BLOCK, _D), lambda i: (i, 0)),
        out_shape=jax.ShapeDtypeStruct((n, _D), jnp.float32),
    )(x, W_i, W_c, W_o, bi, bc, bo, w_co, fc_w, fcb)


# speedup vs baseline: 1.5011x; 1.2487x over previous
import jax
import jax.numpy as jnp
from jax.experimental import pallas as pl

_D = 128
_BLOCK = 5000


def _gclstm_body(x_ref, wi_ref, wc_ref, wo_ref, bchi_ref, bchc_ref, bcho_ref,
                 bi_ref, bc_ref, bo_ref, wco_ref, fcw_ref, fcb_ref, o_ref):
    x = x_ref[...]
    xi = jnp.dot(x, wi_ref[...], preferred_element_type=jnp.float32)
    xc = jnp.dot(x, wc_ref[...], preferred_element_type=jnp.float32)
    xo = jnp.dot(x, wo_ref[...], preferred_element_type=jnp.float32)
    gi = jax.nn.sigmoid(xi + (bchi_ref[...] + bi_ref[...]))
    gt = jnp.tanh(xc + (bchc_ref[...] + bc_ref[...]))
    cn = gi * gt
    go = jax.nn.sigmoid(xo + (bcho_ref[...] + bo_ref[...]) + wco_ref[...] * cn)
    hn = go * jnp.tanh(cn)
    out = jax.lax.dot_general(hn, fcw_ref[...],
                              dimension_numbers=(((1,), (1,)), ((), ())),
                              preferred_element_type=jnp.float32)
    o_ref[...] = out + fcb_ref[...]


def kernel(x, edge_index, batch, W_i, W_f, W_c, W_o, Th_i, Th_f, Th_c, Th_o,
           bch_i, bch_f, bch_c, bch_o, w_ci, w_cf, w_co, b_i, b_f, b_c, b_o,
           fc_w, fc_b):
    n = x.shape[0]
    full = lambda shape: pl.BlockSpec(shape, lambda i: (0,) * len(shape))
    return pl.pallas_call(
        _gclstm_body,
        grid=(n // _BLOCK,),
        in_specs=[
            pl.BlockSpec((_BLOCK, _D), lambda i: (i, 0)),
            full((_D, _D)), full((_D, _D)), full((_D, _D)),
            full((1, _D)), full((1, _D)), full((1, _D)),
            full((1, _D)), full((1, _D)), full((1, _D)),
            full((1, _D)), full((_D, _D)), full((1, _D)),
        ],
        out_specs=pl.BlockSpec((_BLOCK, _D), lambda i: (i, 0)),
        out_shape=jax.ShapeDtypeStruct((n, _D), jnp.float32),
    )(x, W_i, W_c, W_o, bch_i[None, :], bch_c[None, :], bch_o[None, :],
      b_i, b_c, b_o, w_co, fc_w, fc_b[None, :])


# speedup vs baseline: 1.5351x; 1.0226x over previous
import jax
import jax.numpy as jnp
from jax.experimental import pallas as pl

_D = 128
_BLOCK = 2000


def _gclstm_body(x_ref, wi_ref, wc_ref, wo_ref, bchi_ref, bchc_ref, bcho_ref,
                 bi_ref, bc_ref, bo_ref, wco_ref, fcw_ref, fcb_ref, o_ref):
    x = x_ref[...]
    xi = jnp.dot(x, wi_ref[...], preferred_element_type=jnp.float32)
    xc = jnp.dot(x, wc_ref[...], preferred_element_type=jnp.float32)
    xo = jnp.dot(x, wo_ref[...], preferred_element_type=jnp.float32)
    gi = jax.nn.sigmoid(xi + (bchi_ref[...] + bi_ref[...]))
    gt = jnp.tanh(xc + (bchc_ref[...] + bc_ref[...]))
    cn = gi * gt
    go = jax.nn.sigmoid(xo + (bcho_ref[...] + bo_ref[...]) + wco_ref[...] * cn)
    hn = go * jnp.tanh(cn)
    out = jax.lax.dot_general(hn, fcw_ref[...],
                              dimension_numbers=(((1,), (1,)), ((), ())),
                              preferred_element_type=jnp.float32)
    o_ref[...] = out + fcb_ref[...]


def kernel(x, edge_index, batch, W_i, W_f, W_c, W_o, Th_i, Th_f, Th_c, Th_o,
           bch_i, bch_f, bch_c, bch_o, w_ci, w_cf, w_co, b_i, b_f, b_c, b_o,
           fc_w, fc_b):
    n = x.shape[0]
    full = lambda shape: pl.BlockSpec(shape, lambda i: (0,) * len(shape))
    return pl.pallas_call(
        _gclstm_body,
        grid=(n // _BLOCK,),
        in_specs=[
            pl.BlockSpec((_BLOCK, _D), lambda i: (i, 0)),
            full((_D, _D)), full((_D, _D)), full((_D, _D)),
            full((1, _D)), full((1, _D)), full((1, _D)),
            full((1, _D)), full((1, _D)), full((1, _D)),
            full((1, _D)), full((_D, _D)), full((1, _D)),
        ],
        out_specs=pl.BlockSpec((_BLOCK, _D), lambda i: (i, 0)),
        out_shape=jax.ShapeDtypeStruct((n, _D), jnp.float32),
    )(x, W_i, W_c, W_o, bch_i[None, :], bch_c[None, :], bch_o[None, :],
      b_i, b_c, b_o, w_co, fc_w, fc_b[None, :])
